# Initial kernel scaffold; baseline (speedup 1.0000x reference)
import jax
import jax.numpy as jnp
from jax.experimental import pallas as pl


def kernel(hidden_states, expert_indices, routing_weights, W, b):
    raise NotImplementedError("write your pallas kernel here")


# speedup vs baseline: 7.8167x; 7.8167x over previous
import functools

import jax
import jax.numpy as jnp
from jax import lax
from jax.experimental import pallas as pl
from jax.experimental.pallas import tpu as pltpu
from jax.experimental.pallas import tpu_sc as plsc

NUM_E = 64
N_TOK = 8192
D = 768
TM = 128
NB = N_TOK // TM
MAX_ITEMS = NB + NUM_E - 1

NC = 2
NS = 16
NW = NC * NS
ROWS_W = N_TOK // NW
CHUNK = 64
NCHUNK = ROWS_W // CHUNK

def _sc_mesh():
    return plsc.VectorSubcoreMesh(
        core_axis_name="c", subcore_axis_name="s",
        num_cores=NC, num_subcores=NS)


@functools.cache
def _gather_kernel():
    @functools.partial(
        pl.kernel,
        out_type=jax.ShapeDtypeStruct((N_TOK, D), jnp.float32),
        mesh=_sc_mesh(),
        scratch_types=[
            pltpu.VMEM((CHUNK,), jnp.int32),
            pltpu.VMEM((CHUNK, D), jnp.float32),
            pltpu.SemaphoreType.DMA,
        ],
    )
    def _gather_k(h_hbm, idx_hbm, xs_hbm, idx_c, rows, sem):
        wid = lax.axis_index("s") * NC + lax.axis_index("c")
        base = wid * ROWS_W
        for c in range(NCHUNK):
            pltpu.sync_copy(idx_hbm.at[pl.ds(base + c * CHUNK, CHUNK)], idx_c)
            pltpu.async_copy(h_hbm.at[idx_c], rows, sem).wait()
            pltpu.sync_copy(rows, xs_hbm.at[pl.ds(base + c * CHUNK, CHUNK)])

    return _gather_k


@functools.cache
def _scatter_kernel():
    @functools.partial(
        pl.kernel,
        out_type=jax.ShapeDtypeStruct((N_TOK, D), jnp.float32),
        mesh=_sc_mesh(),
        scratch_types=[
            pltpu.VMEM((CHUNK,), jnp.int32),
            pltpu.VMEM((CHUNK, D), jnp.float32),
            pltpu.SemaphoreType.DMA,
        ],
    )
    def _scatter_k(y_hbm, idx_hbm, out_hbm, idx_c, rows, sem):
        wid = lax.axis_index("s") * NC + lax.axis_index("c")
        base = wid * ROWS_W
        for c in range(NCHUNK):
            pltpu.sync_copy(idx_hbm.at[pl.ds(base + c * CHUNK, CHUNK)], idx_c)
            pltpu.sync_copy(y_hbm.at[pl.ds(base + c * CHUNK, CHUNK)], rows)
            pltpu.async_copy(rows, out_hbm.at[idx_c], sem).wait()

    return _scatter_k


def _mm_body(bid_r, eid_r, fst_r, st_r, en_r, x_r, w_r, b_r, rw_r, o_r):
    i = pl.program_id(0)
    xb = x_r[...].astype(jnp.bfloat16)
    wb = w_r[0].astype(jnp.bfloat16)
    acc = jnp.dot(xb, wb, preferred_element_type=jnp.float32)
    acc = (acc + b_r[0]) * rw_r[...][:, :1]
    r0 = bid_r[i] * TM
    rows = r0 + lax.broadcasted_iota(jnp.int32, (TM, 1), 0)
    mask = (rows >= st_r[i]) & (rows < en_r[i])
    prev = jnp.where(fst_r[i] > 0, jnp.zeros_like(acc), o_r[...])
    o_r[...] = jnp.where(mask, acc, prev)


def _grouped_matmul(xs, W, b, rws2, bid, eid, fst, st, en):
    grid_spec = pltpu.PrefetchScalarGridSpec(
        num_scalar_prefetch=5,
        grid=(MAX_ITEMS,),
        in_specs=[
            pl.BlockSpec((TM, D), lambda i, bid, eid, fst, st, en: (bid[i], 0)),
            pl.BlockSpec((1, D, D), lambda i, bid, eid, fst, st, en: (eid[i], 0, 0)),
            pl.BlockSpec((1, 1, D), lambda i, bid, eid, fst, st, en: (eid[i], 0, 0)),
            pl.BlockSpec((TM, 2), lambda i, bid, eid, fst, st, en: (bid[i], 0)),
        ],
        out_specs=pl.BlockSpec((TM, D), lambda i, bid, eid, fst, st, en: (bid[i], 0)),
    )
    return pl.pallas_call(
        _mm_body,
        grid_spec=grid_spec,
        out_shape=jax.ShapeDtypeStruct((N_TOK, D), jnp.float32),
        compiler_params=pltpu.CompilerParams(
            dimension_semantics=("arbitrary",)),
    )(bid, eid, fst, st, en, xs, W, b, rws2)


def _work_items(primary):
    counts = jnp.bincount(primary, length=NUM_E)
    ends = jnp.cumsum(counts)
    starts = ends - counts
    fb = starts // TM
    lb = (ends - 1) // TM
    nb = jnp.where(counts > 0, lb - fb + 1, 0)
    cum_nb = jnp.cumsum(nb)
    excl = cum_nb - nb
    total = cum_nb[-1]
    ii = jnp.minimum(jnp.arange(MAX_ITEMS, dtype=jnp.int32), total - 1)
    eid = jnp.searchsorted(cum_nb, ii, side="right").astype(jnp.int32)
    bid = (fb[eid] + ii - excl[eid]).astype(jnp.int32)
    st = starts[eid].astype(jnp.int32)
    en = ends[eid].astype(jnp.int32)
    fst = jnp.concatenate(
        [jnp.ones((1,), jnp.int32), (bid[1:] != bid[:-1]).astype(jnp.int32)])
    return bid, eid, fst, st, en


def kernel(hidden_states, expert_indices, routing_weights, W, b):
    primary = expert_indices[:, 0].astype(jnp.int32)
    sorted_idx = jnp.argsort(primary, stable=True).astype(jnp.int32)
    bid, eid, fst, st, en = _work_items(primary)
    xs = _gather_kernel()(hidden_states, sorted_idx)
    rws2 = jnp.take(routing_weights, sorted_idx, axis=0)
    y = _grouped_matmul(xs, W, b[:, None, :], rws2, bid, eid, fst, st, en)
    return _scatter_kernel()(y, sorted_idx)
